# baseline (speedup 1.0000x reference)
import functools

import jax
import jax.numpy as jnp
from jax import lax
from jax.experimental import pallas as pl
from jax.experimental.pallas import tpu as pltpu
from jax.experimental.pallas import tpu_sc as plsc

T = 8
B = 32
K = 4
V = 32000
EOS = 2
R = B * K
NROWS = T * R

RB = 128
VC = 32000
NRB = NROWS // RB
NVC = V // VC

_NEG_INF = float("-inf")
_BIG_I32 = 2 ** 30


def _top4_scan_kernel(x_ref, vals_ref, vidx_ref):
    x = x_ref[...]
    iota = jax.lax.broadcasted_iota(jnp.int32, (RB, VC), 1)
    if NVC > 1:
        iota = iota + pl.program_id(1) * VC

    cv = []
    ci = []
    for p in range(4):
        m = jnp.max(x, axis=1, keepdims=True)
        eq = x == m
        am = jnp.min(jnp.where(eq, iota, _BIG_I32), axis=1, keepdims=True)
        cv.append(m)
        ci.append(am)
        if p < 3:
            x = jnp.where(iota == am, _NEG_INF, x)
    chunk_v = jnp.concatenate(cv, axis=1)
    chunk_i = jnp.concatenate(ci, axis=1)

    if NVC == 1:
        vals_ref[...] = chunk_v
        vidx_ref[...] = chunk_i
    else:
        j = pl.program_id(1)

        @pl.when(j == 0)
        def _init():
            vals_ref[...] = chunk_v
            vidx_ref[...] = chunk_i

        @pl.when(j > 0)
        def _merge():
            av = jnp.concatenate([vals_ref[...], chunk_v], axis=1)
            ai = jnp.concatenate([vidx_ref[...], chunk_i], axis=1)
            ov = []
            oi = []
            x8 = av
            for _ in range(4):
                m = jnp.max(x8, axis=1, keepdims=True)
                eq = x8 == m
                am = jnp.min(jnp.where(eq, ai, _BIG_I32), axis=1, keepdims=True)
                ov.append(m)
                oi.append(am)
                x8 = jnp.where(eq & (ai == am), _NEG_INF, x8)
            vals_ref[...] = jnp.concatenate(ov, axis=1)
            vidx_ref[...] = jnp.concatenate(oi, axis=1)


def _beam_kernel(tv_ref, ti_ref, seq_ref, ss_ref):
    lane16 = jax.lax.broadcasted_iota(jnp.int32, (B, 16), 1)
    kk = lane16 // 4
    lane4 = jax.lax.broadcasted_iota(jnp.int32, (B, K), 1)
    neg = jnp.float32(_NEG_INF)

    s = jnp.where(lane4 == 0, jnp.float32(0.0), neg)

    sym_hist = []
    pred_hist = []
    last_scores = None
    for t in range(T):
        tv = tv_ref[t]
        ti = ti_ref[t]
        s16 = jnp.where(kk == 0, s[:, 0:1],
              jnp.where(kk == 1, s[:, 1:2],
              jnp.where(kk == 2, s[:, 2:3], s[:, 3:4])))
        cand = s16 + tv
        flat = kk * V + ti

        sv = []
        sf = []
        x = cand
        for _ in range(4):
            m = jnp.max(x, axis=1, keepdims=True)
            eq = x == m
            am = jnp.min(jnp.where(eq, flat, _BIG_I32), axis=1, keepdims=True)
            sv.append(m)
            sf.append(am)
            x = jnp.where(eq & (flat == am), neg, x)
        sc = jnp.concatenate(sv, axis=1)
        fl = jnp.concatenate(sf, axis=1)
        sym = jnp.remainder(fl, V)
        pk = fl // V
        sym_hist.append(sym)
        pred_hist.append(pk)
        last_scores = sc
        s = jnp.where(sym == EOS, neg, sc)

    pv = []
    pi = []
    x = last_scores
    for _ in range(4):
        m = jnp.max(x, axis=1, keepdims=True)
        eq = x == m
        am = jnp.min(jnp.where(eq, lane4, _BIG_I32), axis=1, keepdims=True)
        pv.append(m)
        pi.append(am)
        x = jnp.where(eq & (lane4 == am), neg, x)
    ss_ref[...] = jnp.concatenate(pv, axis=1)
    tp = jnp.concatenate(pi, axis=1)

    def gather4(val, idx):
        acc = jnp.broadcast_to(val[:, 0:1], (B, K))
        for kslot in range(1, K):
            acc = jnp.where(idx == kslot, val[:, kslot:kslot + 1], acc)
        return acc

    for t in range(T - 1, -1, -1):
        seq_ref[t] = gather4(sym_hist[t], tp)
        tp = gather4(pred_hist[t], tp)


def _top4_of_16(x, tiebreak, neg):
    ms = []
    ams = []
    x = list(x)
    for p in range(4):
        m = x[0]
        a = tiebreak[0]
        for l in range(1, 16):
            better = (x[l] > m) | ((x[l] == m) & (tiebreak[l] < a))
            m = jnp.where(better, x[l], m)
            a = jnp.where(better, tiebreak[l], a)
        ms.append(m)
        ams.append(a)
        if p < 3:
            x = [jnp.where((x[l] == m) & (tiebreak[l] == a), neg, x[l])
                 for l in range(16)]
    return ms, ams


def _sel4(idx, vals):
    return jnp.where(idx == 0, vals[0],
           jnp.where(idx == 1, vals[1],
           jnp.where(idx == 2, vals[2], vals[3])))


def _lanes4_by(idx_vec, scalars):
    return jnp.where(idx_vec == 0, scalars[0],
           jnp.where(idx_vec == 1, scalars[1],
           jnp.where(idx_vec == 2, scalars[2], scalars[3])))


def _lanes4(iota, scalars, fill):
    v = jnp.where(iota == 0, scalars[0],
        jnp.where(iota == 1, scalars[1],
        jnp.where(iota == 2, scalars[2],
        jnp.where(iota == 3, scalars[3], fill))))
    return v


@functools.cache
def _make_beam_sc():
    mesh = plsc.VectorSubcoreMesh(core_axis_name="c", subcore_axis_name="s")

    @functools.partial(
        pl.kernel,
        mesh=mesh,
        out_type=[
            jax.ShapeDtypeStruct((B, T, 16), jnp.int32),
            jax.ShapeDtypeStruct((B, 16), jnp.float32),
        ],
        scratch_types=[
            pltpu.VMEM((T, 16), jnp.float32),
            pltpu.VMEM((T, 16), jnp.int32),
            pltpu.VMEM((T, 16), jnp.int32),
            pltpu.VMEM((16,), jnp.float32),
        ],
    )
    def _beam_sc(tv_hbm, ti_hbm, seq_hbm, ss_hbm, tv_v, ti_v, seq_v, ss_v):
        b = lax.axis_index("s") * 2 + lax.axis_index("c")
        pltpu.sync_copy(tv_hbm.at[b], tv_v)
        pltpu.sync_copy(ti_hbm.at[b], ti_v)

        neg = jnp.float32(_NEG_INF)
        iota = lax.broadcasted_iota(jnp.int32, (16,), 0)

        s = [jnp.float32(0.0), neg, neg, neg]
        sym_hist = []
        pred_hist = []
        last_scores = None
        for t in range(T):
            tv_row = tv_v[t]
            ti_row = ti_v[t]
            cand = [s[l // 4] + tv_row[l] for l in range(16)]
            flat = [(l // 4) * 32768 + ti_row[l] for l in range(16)]
            ms, ams = _top4_of_16(cand, flat, neg)
            syms = [am & 32767 for am in ams]
            pks = [lax.shift_right_logical(am, 15) for am in ams]
            sym_hist.append(syms)
            pred_hist.append(pks)
            last_scores = ms
            s = [jnp.where(sy == EOS, neg, m) for sy, m in zip(syms, ms)]

        idx4 = [jnp.int32(j) for j in range(4)]
        ss, tp = _top4_of_16(last_scores + [neg] * 12,
                             idx4 + [jnp.int32(j) for j in range(4, 16)], neg)
        ss_v[...] = _lanes4(iota, ss, neg)

        for t in range(T - 1, -1, -1):
            cur = [_sel4(tp[j], sym_hist[t]) for j in range(4)]
            tp = [_sel4(tp[j], pred_hist[t]) for j in range(4)]
            seq_v[t] = _lanes4(iota, cur, jnp.int32(0))

        pltpu.sync_copy(seq_v, seq_hbm.at[b])
        pltpu.sync_copy(ss_v, ss_hbm.at[b])

    return _beam_sc


@jax.jit
def kernel(log_probs):
    lp = log_probs.reshape(NROWS, V)

    vals, vidx = pl.pallas_call(
        _top4_scan_kernel,
        grid=(NRB, NVC),
        in_specs=[pl.BlockSpec((RB, VC), lambda i, j: (i, j))],
        out_specs=[
            pl.BlockSpec((RB, 4), lambda i, j: (i, 0)),
            pl.BlockSpec((RB, 4), lambda i, j: (i, 0)),
        ],
        out_shape=[
            jax.ShapeDtypeStruct((NROWS, 4), jnp.float32),
            jax.ShapeDtypeStruct((NROWS, 4), jnp.int32),
        ],
    )(lp)

    tv = vals.reshape(T, B, K * 4).transpose(1, 0, 2)
    ti = vidx.reshape(T, B, K * 4).transpose(1, 0, 2)

    seq_bt, ss_b = _make_beam_sc()(tv, ti)
    sequences = seq_bt[:, :, :K].transpose(1, 0, 2)
    sorted_scores = ss_b[:, :K]

    return sequences, sorted_scores
